# Initial kernel scaffold; baseline (speedup 1.0000x reference)
import jax
import jax.numpy as jnp
from jax.experimental import pallas as pl


def kernel(input_ids, token_type_ids, word_table, pos_table, type_table, W, b, ln_weight, ln_bias):
    raise NotImplementedError("write your pallas kernel here")


# speedup vs baseline: 3.6141x; 3.6141x over previous
import functools

import jax
import jax.numpy as jnp
from jax import lax
from jax.experimental import pallas as pl
from jax.experimental.pallas import tpu as pltpu
from jax.experimental.pallas import tpu_sc as plsc

NC = 2
NS = 16
NW = NC * NS
CHUNK = 128


def _gather_rows(table, ids, emb):
    n = ids.shape[0]
    per_w = n // NW
    ch = per_w // CHUNK
    ids3 = ids.reshape(NW, ch, CHUNK)
    mesh = plsc.VectorSubcoreMesh(core_axis_name="c", subcore_axis_name="s")

    @functools.partial(
        pl.kernel,
        mesh=mesh,
        out_type=jax.ShapeDtypeStruct((n, emb), jnp.float32),
        scratch_types=[
            pltpu.VMEM((ch, CHUNK), jnp.int32),
            pltpu.VMEM((CHUNK, emb), jnp.float32),
            pltpu.VMEM((CHUNK, emb), jnp.float32),
            pltpu.SemaphoreType.DMA,
            pltpu.SemaphoreType.DMA,
        ],
    )
    def k(table_hbm, idx_hbm, out_hbm, idx_v, buf0, buf1, sem0, sem1):
        wid = lax.axis_index("s") * NC + lax.axis_index("c")
        pltpu.sync_copy(idx_hbm.at[wid], idx_v)
        base = wid * per_w
        pltpu.async_copy(table_hbm.at[idx_v.at[0]], buf0, sem0)

        def pair(i, carry):
            j = 2 * i
            pltpu.async_copy(table_hbm.at[idx_v.at[j + 1]], buf1, sem1)
            pltpu.make_async_copy(table_hbm.at[idx_v.at[j]], buf0, sem0).wait()
            pltpu.sync_copy(buf0, out_hbm.at[pl.ds(base + j * CHUNK, CHUNK)])

            @pl.when(j + 2 < ch)
            def _():
                pltpu.async_copy(table_hbm.at[idx_v.at[j + 2]], buf0, sem0)

            pltpu.make_async_copy(table_hbm.at[idx_v.at[j + 1]], buf1, sem1).wait()
            pltpu.sync_copy(buf1, out_hbm.at[pl.ds(base + (j + 1) * CHUNK, CHUNK)])
            return carry

        lax.fori_loop(0, ch // 2, pair, 0)

    return k(table, ids3)


def _project(we, tt3, pos_table, type_table, W, b2, g2, lnb2):
    B, S, emb = we.shape
    hid = W.shape[1]

    def body(we_ref, tt_ref, pos_ref, type_ref, w_ref, b_ref, g_ref, lnb_ref,
             out_ref):
        x = we_ref[0]
        zero = jnp.zeros((1, emb), jnp.float32)
        up = jnp.concatenate([x[1:], zero], axis=0)
        down = jnp.concatenate([zero, x[:-1]], axis=0)
        cat = jnp.concatenate([up, x, down], axis=1)
        acc = jnp.dot(cat, w_ref[...], preferred_element_type=jnp.float32)
        ttf = tt_ref[0].astype(jnp.float32)
        te = type_ref[0][None, :] + ttf * (type_ref[1] - type_ref[0])[None, :]
        out_ref[0] = ((acc + b_ref[...] + pos_ref[...] + te) * g_ref[...]
                      + lnb_ref[...])

    return pl.pallas_call(
        body,
        grid=(B,),
        in_specs=[
            pl.BlockSpec((1, S, emb), lambda i: (i, 0, 0)),
            pl.BlockSpec((1, S, 1), lambda i: (i, 0, 0)),
            pl.BlockSpec((S, hid), lambda i: (0, 0)),
            pl.BlockSpec(type_table.shape, lambda i: (0, 0)),
            pl.BlockSpec(W.shape, lambda i: (0, 0)),
            pl.BlockSpec((1, hid), lambda i: (0, 0)),
            pl.BlockSpec((1, hid), lambda i: (0, 0)),
            pl.BlockSpec((1, hid), lambda i: (0, 0)),
        ],
        out_specs=pl.BlockSpec((1, S, hid), lambda i: (i, 0, 0)),
        out_shape=jax.ShapeDtypeStruct((B, S, hid), jnp.float32),
    )(we, tt3, pos_table, type_table, W, b2, g2, lnb2)


def kernel(input_ids, token_type_ids, word_table, pos_table, type_table, W, b,
           ln_weight, ln_bias):
    B, S = input_ids.shape
    emb = word_table.shape[1]
    hid = W.shape[1]
    ids = input_ids.astype(jnp.int32).reshape(B * S)
    we = _gather_rows(word_table, ids, emb).reshape(B, S, emb)
    tt3 = token_type_ids.astype(jnp.int32).reshape(B, S, 1)
    out = _project(we, tt3, pos_table, type_table, W,
                   b.reshape(1, hid), ln_weight.reshape(1, hid),
                   ln_bias.reshape(1, hid))
    return out


# speedup vs baseline: 3.6449x; 1.0085x over previous
import functools

import jax
import jax.numpy as jnp
from jax import lax
from jax.experimental import pallas as pl
from jax.experimental.pallas import tpu as pltpu
from jax.experimental.pallas import tpu_sc as plsc

NC = 2
NS = 16
NW = NC * NS
CHUNK = 128


def _gather_rows(table, ids, emb):
    n = ids.shape[0]
    per_w = n // NW
    ch = per_w // CHUNK
    ids3 = ids.reshape(NW, ch, CHUNK)
    mesh = plsc.VectorSubcoreMesh(core_axis_name="c", subcore_axis_name="s")

    @functools.partial(
        pl.kernel,
        mesh=mesh,
        out_type=jax.ShapeDtypeStruct((n, emb), jnp.float32),
        scratch_types=[
            pltpu.VMEM((ch, CHUNK), jnp.int32),
            pltpu.VMEM((CHUNK, emb), jnp.float32),
            pltpu.VMEM((CHUNK, emb), jnp.float32),
            pltpu.SemaphoreType.DMA,
            pltpu.SemaphoreType.DMA,
        ],
    )
    def k(table_hbm, idx_hbm, out_hbm, idx_v, buf0, buf1, sem0, sem1):
        wid = lax.axis_index("s") * NC + lax.axis_index("c")
        pltpu.sync_copy(idx_hbm.at[wid], idx_v)
        base = wid * per_w
        pltpu.async_copy(table_hbm.at[idx_v.at[0]], buf0, sem0)

        def pair(i, carry):
            j = 2 * i
            pltpu.async_copy(table_hbm.at[idx_v.at[j + 1]], buf1, sem1)
            pltpu.make_async_copy(table_hbm.at[idx_v.at[j]], buf0, sem0).wait()
            pltpu.sync_copy(buf0, out_hbm.at[pl.ds(base + j * CHUNK, CHUNK)])

            @pl.when(j + 2 < ch)
            def _():
                pltpu.async_copy(table_hbm.at[idx_v.at[j + 2]], buf0, sem0)

            pltpu.make_async_copy(table_hbm.at[idx_v.at[j + 1]], buf1, sem1).wait()
            pltpu.sync_copy(buf1, out_hbm.at[pl.ds(base + (j + 1) * CHUNK, CHUNK)])
            return carry

        lax.fori_loop(0, ch // 2, pair, 0)

    return k(table, ids3)


def _project(we, tt3, pos_table, type_table, W, b2, g2, lnb2):
    B, S, emb = we.shape
    hid = W.shape[1]

    def body(we_ref, tt_ref, pos_ref, type_ref, w_ref, b_ref, g_ref, lnb_ref,
             out_ref):
        x = we_ref[0].astype(jnp.bfloat16)
        zero = jnp.zeros((1, emb), jnp.bfloat16)
        up = jnp.concatenate([x[1:], zero], axis=0)
        down = jnp.concatenate([zero, x[:-1]], axis=0)
        cat = jnp.concatenate([up, x, down], axis=1)
        acc = jnp.dot(cat, w_ref[...], preferred_element_type=jnp.float32)
        ttf = tt_ref[0].astype(jnp.float32)
        te = type_ref[0][None, :] + ttf * (type_ref[1] - type_ref[0])[None, :]
        out_ref[0] = ((acc + b_ref[...] + pos_ref[...] + te) * g_ref[...]
                      + lnb_ref[...])

    return pl.pallas_call(
        body,
        grid=(B,),
        in_specs=[
            pl.BlockSpec((1, S, emb), lambda i: (i, 0, 0)),
            pl.BlockSpec((1, S, 1), lambda i: (i, 0, 0)),
            pl.BlockSpec((S, hid), lambda i: (0, 0)),
            pl.BlockSpec(type_table.shape, lambda i: (0, 0)),
            pl.BlockSpec(W.shape, lambda i: (0, 0)),
            pl.BlockSpec((1, hid), lambda i: (0, 0)),
            pl.BlockSpec((1, hid), lambda i: (0, 0)),
            pl.BlockSpec((1, hid), lambda i: (0, 0)),
        ],
        out_specs=pl.BlockSpec((1, S, hid), lambda i: (i, 0, 0)),
        out_shape=jax.ShapeDtypeStruct((B, S, hid), jnp.float32),
    )(we, tt3, pos_table, type_table, W, b2, g2, lnb2)


def kernel(input_ids, token_type_ids, word_table, pos_table, type_table, W, b,
           ln_weight, ln_bias):
    B, S = input_ids.shape
    emb = word_table.shape[1]
    hid = W.shape[1]
    ids = input_ids.astype(jnp.int32).reshape(B * S)
    we = _gather_rows(word_table, ids, emb).reshape(B, S, emb)
    tt3 = token_type_ids.astype(jnp.int32).reshape(B, S, 1)
    out = _project(we, tt3, pos_table, type_table, W.astype(jnp.bfloat16),
                   b.reshape(1, hid), ln_weight.reshape(1, hid),
                   ln_bias.reshape(1, hid))
    return out


# speedup vs baseline: 3.7123x; 1.0185x over previous
import functools

import jax
import jax.numpy as jnp
from jax import lax
from jax.experimental import pallas as pl
from jax.experimental.pallas import tpu as pltpu
from jax.experimental.pallas import tpu_sc as plsc

NC = 2
NS = 16
NW = NC * NS
CHUNK = 128


def _gather_rows(table, ids, emb):
    n = ids.shape[0]
    per_w = n // NW
    ch = per_w // CHUNK
    ids3 = ids.reshape(NW, ch, CHUNK)
    mesh = plsc.VectorSubcoreMesh(core_axis_name="c", subcore_axis_name="s")

    @functools.partial(
        pl.kernel,
        mesh=mesh,
        out_type=jax.ShapeDtypeStruct((n, emb), jnp.float32),
        scratch_types=[
            pltpu.VMEM((ch, CHUNK), jnp.int32),
            pltpu.VMEM((CHUNK, emb), jnp.float32),
            pltpu.VMEM((CHUNK, emb), jnp.float32),
            pltpu.SemaphoreType.DMA,
            pltpu.SemaphoreType.DMA,
        ],
    )
    def k(table_hbm, idx_hbm, out_hbm, idx_v, buf0, buf1, sem0, sem1):
        wid = lax.axis_index("s") * NC + lax.axis_index("c")
        pltpu.sync_copy(idx_hbm.at[wid], idx_v)
        base = wid * per_w
        pltpu.async_copy(table_hbm.at[idx_v.at[0]], buf0, sem0)

        def pair(i, carry):
            j = 2 * i
            pltpu.async_copy(table_hbm.at[idx_v.at[j + 1]], buf1, sem1)
            pltpu.make_async_copy(table_hbm.at[idx_v.at[j]], buf0, sem0).wait()
            pltpu.sync_copy(buf0, out_hbm.at[pl.ds(base + j * CHUNK, CHUNK)])

            @pl.when(j + 2 < ch)
            def _():
                pltpu.async_copy(table_hbm.at[idx_v.at[j + 2]], buf0, sem0)

            pltpu.make_async_copy(table_hbm.at[idx_v.at[j + 1]], buf1, sem1).wait()
            pltpu.sync_copy(buf1, out_hbm.at[pl.ds(base + (j + 1) * CHUNK, CHUNK)])
            return carry

        lax.fori_loop(0, ch // 2, pair, 0)

    return k(table, ids3)


def _project(we, tt3, p2, dtg, wg):
    B, S, emb = we.shape
    hid = wg.shape[1]

    def body(we_ref, tt_ref, p2_ref, dtg_ref, w_ref, out_ref):
        x = we_ref[0].astype(jnp.bfloat16)
        zero = jnp.zeros((1, emb), jnp.bfloat16)
        up = jnp.concatenate([x[1:], zero], axis=0)
        down = jnp.concatenate([zero, x[:-1]], axis=0)
        cat = jnp.concatenate([up, x, down], axis=1)
        acc = jnp.dot(cat, w_ref[...], preferred_element_type=jnp.float32)
        ttf = tt_ref[0].astype(jnp.float32)
        out_ref[0] = acc + ttf * dtg_ref[...] + p2_ref[...]

    return pl.pallas_call(
        body,
        grid=(B,),
        in_specs=[
            pl.BlockSpec((1, S, emb), lambda i: (i, 0, 0)),
            pl.BlockSpec((1, S, 1), lambda i: (i, 0, 0)),
            pl.BlockSpec((S, hid), lambda i: (0, 0)),
            pl.BlockSpec((1, hid), lambda i: (0, 0)),
            pl.BlockSpec(wg.shape, lambda i: (0, 0)),
        ],
        out_specs=pl.BlockSpec((1, S, hid), lambda i: (i, 0, 0)),
        out_shape=jax.ShapeDtypeStruct((B, S, hid), jnp.float32),
    )(we, tt3, p2, dtg, wg)


def kernel(input_ids, token_type_ids, word_table, pos_table, type_table, W, b,
           ln_weight, ln_bias):
    B, S = input_ids.shape
    emb = word_table.shape[1]
    hid = W.shape[1]
    ids = input_ids.astype(jnp.int32).reshape(B * S)
    we = _gather_rows(word_table, ids, emb).reshape(B, S, emb)
    tt3 = token_type_ids.astype(jnp.int32).reshape(B, S, 1)
    g = ln_weight.reshape(1, hid)
    p2 = (pos_table + b.reshape(1, hid) + type_table[0].reshape(1, hid)) * g \
        + ln_bias.reshape(1, hid)
    dtg = ((type_table[1] - type_table[0]).reshape(1, hid) * g)
    wg = (W * ln_weight.reshape(1, hid)).astype(jnp.bfloat16)
    out = _project(we, tt3, p2, dtg, wg)
    return out


# speedup vs baseline: 4.7550x; 1.2809x over previous
import functools

import jax
import jax.numpy as jnp
from jax import lax
from jax.experimental import pallas as pl
from jax.experimental.pallas import tpu as pltpu
from jax.experimental.pallas import tpu_sc as plsc

NC = 2
NS = 16
NW = NC * NS
CHUNK = 128


def _gather_rows(table, ids, emb):
    n = ids.shape[0]
    per_w = n // NW
    ch = per_w // CHUNK
    ids3 = ids.reshape(NW, ch, CHUNK)
    mesh = plsc.VectorSubcoreMesh(core_axis_name="c", subcore_axis_name="s")

    @functools.partial(
        pl.kernel,
        mesh=mesh,
        out_type=jax.ShapeDtypeStruct((n, emb), jnp.float32),
        scratch_types=[
            pltpu.VMEM((ch, CHUNK), jnp.int32),
            pltpu.VMEM((CHUNK, emb), jnp.float32),
            pltpu.VMEM((CHUNK, emb), jnp.float32),
            pltpu.SemaphoreType.DMA,
            pltpu.SemaphoreType.DMA,
        ],
    )
    def k(table_hbm, idx_hbm, out_hbm, idx_v, buf0, buf1, sem0, sem1):
        wid = lax.axis_index("s") * NC + lax.axis_index("c")
        pltpu.sync_copy(idx_hbm.at[wid], idx_v)
        base = wid * per_w
        pltpu.async_copy(table_hbm.at[idx_v.at[0]], buf0, sem0)

        def pair(i, carry):
            j = 2 * i
            pltpu.async_copy(table_hbm.at[idx_v.at[j + 1]], buf1, sem1)
            pltpu.make_async_copy(table_hbm.at[idx_v.at[j]], buf0, sem0).wait()
            pltpu.sync_copy(buf0, out_hbm.at[pl.ds(base + j * CHUNK, CHUNK)])

            @pl.when(j + 2 < ch)
            def _():
                pltpu.async_copy(table_hbm.at[idx_v.at[j + 2]], buf0, sem0)

            pltpu.make_async_copy(table_hbm.at[idx_v.at[j + 1]], buf1, sem1).wait()
            pltpu.sync_copy(buf1, out_hbm.at[pl.ds(base + (j + 1) * CHUNK, CHUNK)])
            return carry

        lax.fori_loop(0, ch // 2, pair, 0)

    return k(table, ids3)


def _project(we, tt3, p2, dtg, wg):
    B, S, emb = we.shape
    hid = wg.shape[1]

    BB = 2

    def body(we_ref, tt_ref, p2_ref, dtg_ref, w_ref, out_ref):
        zero = jnp.zeros((1, emb), jnp.bfloat16)
        for q in range(BB):
            x = we_ref[q].astype(jnp.bfloat16)
            up = jnp.concatenate([x[1:], zero], axis=0)
            down = jnp.concatenate([zero, x[:-1]], axis=0)
            cat = jnp.concatenate([up, x, down], axis=1)
            acc = jnp.dot(cat, w_ref[...], preferred_element_type=jnp.float32)
            ttf = tt_ref[q].astype(jnp.float32)
            out_ref[q] = acc + ttf * dtg_ref[...] + p2_ref[...]

    return pl.pallas_call(
        body,
        grid=(B // BB,),
        in_specs=[
            pl.BlockSpec((BB, S, emb), lambda i: (i, 0, 0)),
            pl.BlockSpec((BB, S, 1), lambda i: (i, 0, 0)),
            pl.BlockSpec((S, hid), lambda i: (0, 0)),
            pl.BlockSpec((1, hid), lambda i: (0, 0)),
            pl.BlockSpec(wg.shape, lambda i: (0, 0)),
        ],
        out_specs=pl.BlockSpec((BB, S, hid), lambda i: (i, 0, 0)),
        out_shape=jax.ShapeDtypeStruct((B, S, hid), jnp.float32),
    )(we, tt3, p2, dtg, wg)


def kernel(input_ids, token_type_ids, word_table, pos_table, type_table, W, b,
           ln_weight, ln_bias):
    B, S = input_ids.shape
    emb = word_table.shape[1]
    hid = W.shape[1]
    ids = input_ids.astype(jnp.int32).reshape(B * S)
    we = _gather_rows(word_table, ids, emb).reshape(B, S, emb)
    tt3 = token_type_ids.astype(jnp.int32).reshape(B, S, 1)
    g = ln_weight.reshape(1, hid)
    p2 = (pos_table + b.reshape(1, hid) + type_table[0].reshape(1, hid)) * g \
        + ln_bias.reshape(1, hid)
    dtg = ((type_table[1] - type_table[0]).reshape(1, hid) * g)
    wg = (W * ln_weight.reshape(1, hid)).astype(jnp.bfloat16)
    out = _project(we, tt3, p2, dtg, wg)
    return out


# speedup vs baseline: 5.6355x; 1.1852x over previous
import functools

import jax
import jax.numpy as jnp
from jax import lax
from jax.experimental import pallas as pl
from jax.experimental.pallas import tpu as pltpu
from jax.experimental.pallas import tpu_sc as plsc

NC = 2
NS = 16
NW = NC * NS
CHUNK = 128


def _gather_rows(table, ids, emb):
    n = ids.shape[0]
    per_w = n // NW
    ch = per_w // CHUNK
    ids3 = ids.reshape(NW, ch, CHUNK)
    mesh = plsc.VectorSubcoreMesh(core_axis_name="c", subcore_axis_name="s")

    @functools.partial(
        pl.kernel,
        mesh=mesh,
        out_type=jax.ShapeDtypeStruct((n, emb), jnp.float32),
        scratch_types=[
            pltpu.VMEM((ch, CHUNK), jnp.int32),
            pltpu.VMEM((CHUNK, emb), jnp.float32),
            pltpu.VMEM((CHUNK, emb), jnp.float32),
            pltpu.SemaphoreType.DMA,
            pltpu.SemaphoreType.DMA,
        ],
    )
    def k(table_hbm, idx_hbm, out_hbm, idx_v, buf0, buf1, sem0, sem1):
        wid = lax.axis_index("s") * NC + lax.axis_index("c")
        pltpu.sync_copy(idx_hbm.at[wid], idx_v)
        base = wid * per_w
        pltpu.async_copy(table_hbm.at[idx_v.at[0]], buf0, sem0)

        def pair(i, carry):
            j = 2 * i
            pltpu.async_copy(table_hbm.at[idx_v.at[j + 1]], buf1, sem1)
            pltpu.make_async_copy(table_hbm.at[idx_v.at[j]], buf0, sem0).wait()
            pltpu.sync_copy(buf0, out_hbm.at[pl.ds(base + j * CHUNK, CHUNK)])

            @pl.when(j + 2 < ch)
            def _():
                pltpu.async_copy(table_hbm.at[idx_v.at[j + 2]], buf0, sem0)

            pltpu.make_async_copy(table_hbm.at[idx_v.at[j + 1]], buf1, sem1).wait()
            pltpu.sync_copy(buf1, out_hbm.at[pl.ds(base + (j + 1) * CHUNK, CHUNK)])
            return carry

        lax.fori_loop(0, ch // 2, pair, 0)

    return k(table, ids3)


def _project(we, tt3, p2, dtg, wg):
    B, S, emb = we.shape
    hid = wg.shape[1]

    BB = 4

    def body(we_ref, tt_ref, p2_ref, dtg_ref, w_ref, out_ref):
        zero = jnp.zeros((1, emb), jnp.bfloat16)
        for q in range(BB):
            x = we_ref[q].astype(jnp.bfloat16)
            up = jnp.concatenate([x[1:], zero], axis=0)
            down = jnp.concatenate([zero, x[:-1]], axis=0)
            cat = jnp.concatenate([up, x, down], axis=1)
            acc = jnp.dot(cat, w_ref[...], preferred_element_type=jnp.float32)
            ttf = tt_ref[q].astype(jnp.float32)
            out_ref[q] = acc + ttf * dtg_ref[...] + p2_ref[...]

    return pl.pallas_call(
        body,
        grid=(B // BB,),
        in_specs=[
            pl.BlockSpec((BB, S, emb), lambda i: (i, 0, 0)),
            pl.BlockSpec((BB, S, 1), lambda i: (i, 0, 0)),
            pl.BlockSpec((S, hid), lambda i: (0, 0)),
            pl.BlockSpec((1, hid), lambda i: (0, 0)),
            pl.BlockSpec(wg.shape, lambda i: (0, 0)),
        ],
        out_specs=pl.BlockSpec((BB, S, hid), lambda i: (i, 0, 0)),
        out_shape=jax.ShapeDtypeStruct((B, S, hid), jnp.float32),
    )(we, tt3, p2, dtg, wg)


def kernel(input_ids, token_type_ids, word_table, pos_table, type_table, W, b,
           ln_weight, ln_bias):
    B, S = input_ids.shape
    emb = word_table.shape[1]
    hid = W.shape[1]
    ids = input_ids.astype(jnp.int32).reshape(B * S)
    we = _gather_rows(word_table, ids, emb).reshape(B, S, emb)
    tt3 = token_type_ids.astype(jnp.int32).reshape(B, S, 1)
    g = ln_weight.reshape(1, hid)
    p2 = (pos_table + b.reshape(1, hid) + type_table[0].reshape(1, hid)) * g \
        + ln_bias.reshape(1, hid)
    dtg = ((type_table[1] - type_table[0]).reshape(1, hid) * g)
    wg = (W * ln_weight.reshape(1, hid)).astype(jnp.bfloat16)
    out = _project(we, tt3, p2, dtg, wg)
    return out


# speedup vs baseline: 6.1431x; 1.0901x over previous
import functools

import jax
import jax.numpy as jnp
from jax import lax
from jax.experimental import pallas as pl
from jax.experimental.pallas import tpu as pltpu
from jax.experimental.pallas import tpu_sc as plsc

NC = 2
NS = 16
NW = NC * NS
CHUNK = 128


def _gather_rows(table, ids, emb):
    n = ids.shape[0]
    per_w = n // NW
    ch = per_w // CHUNK
    ids3 = ids.reshape(NW, ch, CHUNK)
    mesh = plsc.VectorSubcoreMesh(core_axis_name="c", subcore_axis_name="s")

    @functools.partial(
        pl.kernel,
        mesh=mesh,
        out_type=jax.ShapeDtypeStruct((n, emb), jnp.float32),
        scratch_types=[
            pltpu.VMEM((ch, CHUNK), jnp.int32),
            pltpu.VMEM((CHUNK, emb), jnp.float32),
            pltpu.VMEM((CHUNK, emb), jnp.float32),
            pltpu.SemaphoreType.DMA,
            pltpu.SemaphoreType.DMA,
        ],
    )
    def k(table_hbm, idx_hbm, out_hbm, idx_v, buf0, buf1, sem0, sem1):
        wid = lax.axis_index("s") * NC + lax.axis_index("c")
        pltpu.sync_copy(idx_hbm.at[wid], idx_v)
        base = wid * per_w
        pltpu.async_copy(table_hbm.at[idx_v.at[0]], buf0, sem0)

        def pair(i, carry):
            j = 2 * i
            pltpu.async_copy(table_hbm.at[idx_v.at[j + 1]], buf1, sem1)
            pltpu.make_async_copy(table_hbm.at[idx_v.at[j]], buf0, sem0).wait()
            pltpu.sync_copy(buf0, out_hbm.at[pl.ds(base + j * CHUNK, CHUNK)])

            @pl.when(j + 2 < ch)
            def _():
                pltpu.async_copy(table_hbm.at[idx_v.at[j + 2]], buf0, sem0)

            pltpu.make_async_copy(table_hbm.at[idx_v.at[j + 1]], buf1, sem1).wait()
            pltpu.sync_copy(buf1, out_hbm.at[pl.ds(base + (j + 1) * CHUNK, CHUNK)])
            return carry

        lax.fori_loop(0, ch // 2, pair, 0)

    return k(table, ids3)


def _project(we, tt3, p2, dtg, wg):
    B, S, emb = we.shape
    hid = wg.shape[1]

    BB = 8

    def body(we_ref, tt_ref, p2_ref, dtg_ref, w_ref, out_ref):
        zero = jnp.zeros((1, emb), jnp.bfloat16)
        for q in range(BB):
            x = we_ref[q].astype(jnp.bfloat16)
            up = jnp.concatenate([x[1:], zero], axis=0)
            down = jnp.concatenate([zero, x[:-1]], axis=0)
            cat = jnp.concatenate([up, x, down], axis=1)
            acc = jnp.dot(cat, w_ref[...], preferred_element_type=jnp.float32)
            ttf = tt_ref[q].astype(jnp.float32)
            out_ref[q] = acc + ttf * dtg_ref[...] + p2_ref[...]

    return pl.pallas_call(
        body,
        grid=(B // BB,),
        in_specs=[
            pl.BlockSpec((BB, S, emb), lambda i: (i, 0, 0)),
            pl.BlockSpec((BB, S, 1), lambda i: (i, 0, 0)),
            pl.BlockSpec((S, hid), lambda i: (0, 0)),
            pl.BlockSpec((1, hid), lambda i: (0, 0)),
            pl.BlockSpec(wg.shape, lambda i: (0, 0)),
        ],
        out_specs=pl.BlockSpec((BB, S, hid), lambda i: (i, 0, 0)),
        out_shape=jax.ShapeDtypeStruct((B, S, hid), jnp.float32),
    )(we, tt3, p2, dtg, wg)


def kernel(input_ids, token_type_ids, word_table, pos_table, type_table, W, b,
           ln_weight, ln_bias):
    B, S = input_ids.shape
    emb = word_table.shape[1]
    hid = W.shape[1]
    ids = input_ids.astype(jnp.int32).reshape(B * S)
    we = _gather_rows(word_table, ids, emb).reshape(B, S, emb)
    tt3 = token_type_ids.astype(jnp.int32).reshape(B, S, 1)
    g = ln_weight.reshape(1, hid)
    p2 = (pos_table + b.reshape(1, hid) + type_table[0].reshape(1, hid)) * g \
        + ln_bias.reshape(1, hid)
    dtg = ((type_table[1] - type_table[0]).reshape(1, hid) * g)
    wg = (W * ln_weight.reshape(1, hid)).astype(jnp.bfloat16)
    out = _project(we, tt3, p2, dtg, wg)
    return out


# speedup vs baseline: 6.2628x; 1.0195x over previous
import functools

import jax
import jax.numpy as jnp
from jax import lax
from jax.experimental import pallas as pl
from jax.experimental.pallas import tpu as pltpu
from jax.experimental.pallas import tpu_sc as plsc

NC = 2
NS = 16
NW = NC * NS
CHUNK = 128


def _gather_rows(table, ids, emb):
    n = ids.shape[0]
    per_w = n // NW
    ch = per_w // CHUNK
    ids3 = ids.reshape(NW, ch, CHUNK)
    mesh = plsc.VectorSubcoreMesh(core_axis_name="c", subcore_axis_name="s")

    @functools.partial(
        pl.kernel,
        mesh=mesh,
        out_type=jax.ShapeDtypeStruct((n, emb), jnp.float32),
        scratch_types=[
            pltpu.VMEM((ch, CHUNK), jnp.int32),
            pltpu.VMEM((CHUNK, emb), jnp.float32),
            pltpu.VMEM((CHUNK, emb), jnp.float32),
            pltpu.SemaphoreType.DMA,
            pltpu.SemaphoreType.DMA,
        ],
    )
    def k(table_hbm, idx_hbm, out_hbm, idx_v, buf0, buf1, sem0, sem1):
        wid = lax.axis_index("s") * NC + lax.axis_index("c")
        pltpu.sync_copy(idx_hbm.at[wid], idx_v)
        base = wid * per_w
        pltpu.async_copy(table_hbm.at[idx_v.at[0]], buf0, sem0)

        def pair(i, carry):
            j = 2 * i
            pltpu.async_copy(table_hbm.at[idx_v.at[j + 1]], buf1, sem1)
            pltpu.make_async_copy(table_hbm.at[idx_v.at[j]], buf0, sem0).wait()
            pltpu.sync_copy(buf0, out_hbm.at[pl.ds(base + j * CHUNK, CHUNK)])

            @pl.when(j + 2 < ch)
            def _():
                pltpu.async_copy(table_hbm.at[idx_v.at[j + 2]], buf0, sem0)

            pltpu.make_async_copy(table_hbm.at[idx_v.at[j + 1]], buf1, sem1).wait()
            pltpu.sync_copy(buf1, out_hbm.at[pl.ds(base + (j + 1) * CHUNK, CHUNK)])
            return carry

        lax.fori_loop(0, ch // 2, pair, 0)

    return k(table, ids3)


def _project(we, tt3, p2, dtg, wg):
    B, S, emb = we.shape
    hid = wg.shape[1]

    BB = 16

    def body(we_ref, tt_ref, p2_ref, dtg_ref, w_ref, out_ref):
        zero = jnp.zeros((1, emb), jnp.bfloat16)
        for q in range(BB):
            x = we_ref[q].astype(jnp.bfloat16)
            up = jnp.concatenate([x[1:], zero], axis=0)
            down = jnp.concatenate([zero, x[:-1]], axis=0)
            cat = jnp.concatenate([up, x, down], axis=1)
            acc = jnp.dot(cat, w_ref[...], preferred_element_type=jnp.float32)
            ttf = tt_ref[q].astype(jnp.float32)
            out_ref[q] = acc + ttf * dtg_ref[...] + p2_ref[...]

    return pl.pallas_call(
        body,
        grid=(B // BB,),
        in_specs=[
            pl.BlockSpec((BB, S, emb), lambda i: (i, 0, 0)),
            pl.BlockSpec((BB, S, 1), lambda i: (i, 0, 0)),
            pl.BlockSpec((S, hid), lambda i: (0, 0)),
            pl.BlockSpec((1, hid), lambda i: (0, 0)),
            pl.BlockSpec(wg.shape, lambda i: (0, 0)),
        ],
        out_specs=pl.BlockSpec((BB, S, hid), lambda i: (i, 0, 0)),
        out_shape=jax.ShapeDtypeStruct((B, S, hid), jnp.float32),
    )(we, tt3, p2, dtg, wg)


def kernel(input_ids, token_type_ids, word_table, pos_table, type_table, W, b,
           ln_weight, ln_bias):
    B, S = input_ids.shape
    emb = word_table.shape[1]
    hid = W.shape[1]
    ids = input_ids.astype(jnp.int32).reshape(B * S)
    we = _gather_rows(word_table, ids, emb).reshape(B, S, emb)
    tt3 = token_type_ids.astype(jnp.int32).reshape(B, S, 1)
    g = ln_weight.reshape(1, hid)
    p2 = (pos_table + b.reshape(1, hid) + type_table[0].reshape(1, hid)) * g \
        + ln_bias.reshape(1, hid)
    dtg = ((type_table[1] - type_table[0]).reshape(1, hid) * g)
    wg = (W * ln_weight.reshape(1, hid)).astype(jnp.bfloat16)
    out = _project(we, tt3, p2, dtg, wg)
    return out
